# Initial kernel scaffold; baseline (speedup 1.0000x reference)
import jax
import jax.numpy as jnp
from jax.experimental import pallas as pl


def kernel(x, edge_index, W, b):
    raise NotImplementedError("write your pallas kernel here")


# speedup vs baseline: 7.9619x; 7.9619x over previous
import functools

import jax
import jax.numpy as jnp
from jax import lax
from jax.experimental import pallas as pl
from jax.experimental.pallas import tpu as pltpu
from jax.experimental.pallas import tpu_sc as plsc

N = 10000
E = 320000
D = 128
KSTEPS = 10
ALPHA = 0.1

NCORE = 2
NSUB = 16
NW = NCORE * NSUB
CH = 128
NCHUNK = 79
EPT = NCHUNK * CH
EPAD = EPT * NW
NPAD = 10240
RPT = NPAD // NSUB
DW = 16

_mesh = plsc.VectorSubcoreMesh(
    core_axis_name="c", subcore_axis_name="s",
    num_cores=NCORE, num_subcores=NSUB)


def _sc_scatter_body(y_hbm, src_hbm, dst_hbm, zrows_hbm, out_hbm,
                     src_v, dst_v, rows_v, agg_sh):
  cid = lax.axis_index("c")
  sid = lax.axis_index("s")
  wid = cid * NSUB + sid
  pltpu.sync_copy(src_hbm.at[wid], src_v)
  pltpu.sync_copy(dst_hbm.at[wid], dst_v)
  pltpu.sync_copy(zrows_hbm, agg_sh.at[pl.ds(sid * RPT, RPT)])
  plsc.subcore_barrier()

  def step(j, _):
    pltpu.sync_copy(y_hbm.at[src_v.at[j]], rows_v)
    pltpu.sync_copy(rows_v, agg_sh.at[dst_v.at[j]], add=True)
    return ()

  lax.fori_loop(0, NCHUNK, step, (), unroll=False)
  plsc.subcore_barrier()
  pltpu.sync_copy(agg_sh.at[pl.ds(sid * RPT, RPT)], out_hbm.at[cid, sid])


def _make_sc_scatter(interpret=False):
  return pl.kernel(
      _sc_scatter_body,
      out_type=jax.ShapeDtypeStruct((NCORE, NSUB, RPT, D), jnp.float32),
      mesh=_mesh,
      scratch_types=[
          pltpu.VMEM((NCHUNK, CH), jnp.int32),
          pltpu.VMEM((NCHUNK, CH), jnp.int32),
          pltpu.VMEM((CH, D), jnp.float32),
          pltpu.VMEM_SHARED((NPAD, D), jnp.float32),
      ],
      interpret=interpret,
  )


_sc_scatter = _make_sc_scatter()


def _sc_degree_body(dst_hbm, ones_hbm, zrows_hbm, out_hbm, dst_v, ones_v, deg_sh):
  cid = lax.axis_index("c")
  sid = lax.axis_index("s")
  wid = cid * NSUB + sid
  pltpu.sync_copy(dst_hbm.at[wid], dst_v)
  pltpu.sync_copy(ones_hbm, ones_v)
  pltpu.sync_copy(zrows_hbm, deg_sh.at[pl.ds(sid * RPT, RPT)])
  plsc.subcore_barrier()

  def step(j, _):
    pltpu.sync_copy(ones_v, deg_sh.at[dst_v.at[j]], add=True)
    return ()

  lax.fori_loop(0, NCHUNK, step, (), unroll=False)
  plsc.subcore_barrier()
  pltpu.sync_copy(deg_sh.at[pl.ds(sid * RPT, RPT)], out_hbm.at[cid, sid])


def _make_sc_degree(interpret=False):
  return pl.kernel(
      _sc_degree_body,
      out_type=jax.ShapeDtypeStruct((NCORE, NSUB, RPT, D), jnp.float32),
      mesh=_mesh,
      scratch_types=[
          pltpu.VMEM((NCHUNK, CH), jnp.int32),
          pltpu.VMEM((CH, D), jnp.float32),
          pltpu.VMEM_SHARED((NPAD, D), jnp.float32),
      ],
      interpret=interpret,
  )


_sc_degree = _make_sc_degree()


def _mm_body(x_ref, w_ref, b_ref, o_ref):
  o_ref[...] = jnp.maximum(
      jnp.dot(x_ref[...], w_ref[...], preferred_element_type=jnp.float32)
      + b_ref[...], 0.0)


def _matmul_relu(x, W, b):
  return pl.pallas_call(
      _mm_body,
      out_shape=jax.ShapeDtypeStruct((N, D), jnp.float32),
  )(x, W, b.reshape(1, D))


BR = 1000


def _prep_body(h_ref, cnt_ref, y0_ref, b9_ref, bk_ref, s9_ref, sk_ref):
  deg = 1.0 + cnt_ref[0] + cnt_ref[1]
  dinv = lax.rsqrt(deg)
  h = h_ref[...]
  y0 = dinv * h
  y0_ref[...] = y0
  b9_ref[...] = ALPHA * y0
  bk_ref[...] = ALPHA * h
  s9_ref[...] = (1.0 - ALPHA) / deg
  sk_ref[...] = (1.0 - ALPHA) * dinv


def _prep(h, cnt):
  grid = (N // BR,)
  return pl.pallas_call(
      _prep_body,
      grid=grid,
      in_specs=[
          pl.BlockSpec((BR, D), lambda i: (i, 0)),
          pl.BlockSpec((NCORE, BR, 1), lambda i: (0, i, 0)),
      ],
      out_specs=[
          pl.BlockSpec((BR, D), lambda i: (i, 0)),
          pl.BlockSpec((BR, D), lambda i: (i, 0)),
          pl.BlockSpec((BR, D), lambda i: (i, 0)),
          pl.BlockSpec((BR, 1), lambda i: (i, 0)),
          pl.BlockSpec((BR, 1), lambda i: (i, 0)),
      ],
      out_shape=[
          jax.ShapeDtypeStruct((N, D), jnp.float32),
          jax.ShapeDtypeStruct((N, D), jnp.float32),
          jax.ShapeDtypeStruct((N, D), jnp.float32),
          jax.ShapeDtypeStruct((N, 1), jnp.float32),
          jax.ShapeDtypeStruct((N, 1), jnp.float32),
      ],
  )(h, cnt)


def _blend_body(a_ref, y_ref, s_ref, base_ref, o_ref):
  o_ref[...] = (s_ref[...] * (a_ref[0] + a_ref[1] + y_ref[...])
                + base_ref[...])


def _blend(aggs, y, s, base):
  grid = (N // BR,)
  return pl.pallas_call(
      _blend_body,
      grid=grid,
      in_specs=[
          pl.BlockSpec((NCORE, BR, D), lambda i: (0, i, 0)),
          pl.BlockSpec((BR, D), lambda i: (i, 0)),
          pl.BlockSpec((BR, 1), lambda i: (i, 0)),
          pl.BlockSpec((BR, D), lambda i: (i, 0)),
      ],
      out_specs=pl.BlockSpec((BR, D), lambda i: (i, 0)),
      out_shape=jax.ShapeDtypeStruct((N, D), jnp.float32),
  )(aggs, y, s, base)


def kernel(x, edge_index, W, b):
  src = edge_index[0]
  dst = edge_index[1]
  pad = EPAD - E
  src_r = jnp.concatenate(
      [src, jnp.zeros((pad,), jnp.int32)]).reshape(NW, NCHUNK, CH)
  dst_r = jnp.concatenate(
      [dst, jnp.full((pad,), N, jnp.int32)]).reshape(NW, NCHUNK, CH)
  zrows = jnp.zeros((RPT, D), jnp.float32)
  ones_rows = jnp.ones((CH, D), jnp.float32)

  h = _matmul_relu(x, W, b)
  cnt = _sc_degree(dst_r, ones_rows, zrows)
  cnt = cnt.reshape(NCORE, NPAD, D)[:, :N, 0:1]
  y0, b9, bk, s9, sk = _prep(h, cnt)

  y = y0
  for t in range(KSTEPS):
    aggs = _sc_scatter(y, src_r, dst_r, zrows)
    aggs = aggs.reshape(NCORE, NPAD, D)
    if t < KSTEPS - 1:
      y = _blend(aggs, y, s9, b9)
    else:
      y = _blend(aggs, y, sk, bk)
  return y
